# baseline (speedup 1.0000x reference)
import jax
import jax.numpy as jnp
from jax.experimental import pallas as pl
from jax.experimental.pallas import tpu as pltpu

B, S, C = 4, 8192, 2048
SB = 512
CB = 512


G = SB // 8


def _scan_body(x_ref, o_ref, carry_ref):
    s = pl.program_id(2)

    @pl.when(s == 0)
    def _():
        carry_ref[...] = jnp.zeros_like(carry_ref)

    blk = x_ref[0].reshape(G, 8, CB)
    for d in (1, 2, 4):
        pad = jnp.zeros((G, d, CB), blk.dtype)
        blk = blk + jnp.concatenate([pad, blk[:, : 8 - d, :]], axis=1)
    carry = carry_ref[...]
    for g in range(G):
        o_ref[0, 8 * g : 8 * (g + 1), :] = blk[g] + carry
        carry = carry + blk[g, 7:8, :]
    carry_ref[...] = carry


def kernel(x):
    grid = (B, C // CB, S // SB)
    spec = pl.BlockSpec((1, SB, CB), lambda b, c, s: (b, s, c))
    return pl.pallas_call(
        _scan_body,
        grid=grid,
        in_specs=[spec],
        out_specs=spec,
        out_shape=jax.ShapeDtypeStruct((B, S, C), jnp.float32),
        scratch_shapes=[pltpu.VMEM((1, CB), jnp.float32)],
        compiler_params=pltpu.CompilerParams(
            dimension_semantics=("parallel", "parallel", "arbitrary"),
        ),
    )(x)


# speedup vs baseline: 1.5902x; 1.5902x over previous
import jax
import jax.numpy as jnp
from jax.experimental import pallas as pl
from jax.experimental.pallas import tpu as pltpu

B, S, C = 4, 8192, 2048
SB = 512
CB = 2048


G = SB // 8


def _scan_body(x_ref, o_ref, carry_ref):
    s = pl.program_id(2)

    @pl.when(s == 0)
    def _():
        carry_ref[...] = jnp.zeros_like(carry_ref)

    blk = x_ref[0].reshape(G, 8, CB)
    for d in (1, 2, 4):
        pad = jnp.zeros((G, d, CB), blk.dtype)
        blk = blk + jnp.concatenate([pad, blk[:, : 8 - d, :]], axis=1)
    carry = carry_ref[...]
    for g in range(G):
        o_ref[0, 8 * g : 8 * (g + 1), :] = blk[g] + carry
        carry = carry + blk[g, 7:8, :]
    carry_ref[...] = carry


def kernel(x):
    grid = (B, C // CB, S // SB)
    spec = pl.BlockSpec((1, SB, CB), lambda b, c, s: (b, s, c))
    return pl.pallas_call(
        _scan_body,
        grid=grid,
        in_specs=[spec],
        out_specs=spec,
        out_shape=jax.ShapeDtypeStruct((B, S, C), jnp.float32),
        scratch_shapes=[pltpu.VMEM((1, CB), jnp.float32)],
        compiler_params=pltpu.CompilerParams(
            dimension_semantics=("parallel", "parallel", "arbitrary"),
        ),
    )(x)


# speedup vs baseline: 1.7237x; 1.0839x over previous
import jax
import jax.numpy as jnp
from jax.experimental import pallas as pl
from jax.experimental.pallas import tpu as pltpu

B, S, C = 4, 8192, 2048
SB = 1024
CB = 2048


G = SB // 8


def _scan_body(x_ref, o_ref, carry_ref):
    s = pl.program_id(2)

    @pl.when(s == 0)
    def _():
        carry_ref[...] = jnp.zeros_like(carry_ref)

    blk = x_ref[0].reshape(G, 8, CB)
    for d in (1, 2, 4):
        pad = jnp.zeros((G, d, CB), blk.dtype)
        blk = blk + jnp.concatenate([pad, blk[:, : 8 - d, :]], axis=1)
    carry = carry_ref[...]
    for g in range(G):
        o_ref[0, 8 * g : 8 * (g + 1), :] = blk[g] + carry
        carry = carry + blk[g, 7:8, :]
    carry_ref[...] = carry


def kernel(x):
    grid = (B, C // CB, S // SB)
    spec = pl.BlockSpec((1, SB, CB), lambda b, c, s: (b, s, c))
    return pl.pallas_call(
        _scan_body,
        grid=grid,
        in_specs=[spec],
        out_specs=spec,
        out_shape=jax.ShapeDtypeStruct((B, S, C), jnp.float32),
        scratch_shapes=[pltpu.VMEM((1, CB), jnp.float32)],
        compiler_params=pltpu.CompilerParams(
            dimension_semantics=("parallel", "parallel", "arbitrary"),
        ),
    )(x)
